# baseline (speedup 1.0000x reference)
import functools

import jax
import jax.numpy as jnp
from jax import lax
from jax.experimental import pallas as pl
from jax.experimental.pallas import tpu as pltpu
from jax.experimental.pallas import tpu_sc as plsc

_VOCAB = 1000000
_DIM = 32
_BATCH = 4096
_SEQ = 50
_PAD_IDX = 0

_NW = 32
_LANES = 16

_SW = 768
_NSLAB = _VOCAB // _SW
_TAIL = 128
_TAIL0 = _VOCAB - _TAIL

_BL = 512
_NU = _SEQ * (_BATCH // _BL)
_NG = _BL // _LANES


def _sc_linearize(t_t, t_tail):
  mesh = plsc.VectorSubcoreMesh(core_axis_name="c", subcore_axis_name="s")

  @functools.partial(
      pl.kernel,
      mesh=mesh,
      out_type=jax.ShapeDtypeStruct((_DIM * _VOCAB,), jnp.float32),
      scratch_types=[
          pltpu.VMEM((_DIM, _SW), jnp.float32),
          pltpu.VMEM((_DIM, _SW), jnp.float32),
          pltpu.VMEM((_DIM * _SW,), jnp.float32),
          pltpu.VMEM((_DIM * _SW,), jnp.float32),
          pltpu.SemaphoreType.DMA,
          pltpu.SemaphoreType.DMA,
      ],
  )
  def k(t_h, tail_h, out_h, slab_a, slab_b, flat_a, flat_b, semr, semw):
    wid = lax.axis_index("s") * 2 + lax.axis_index("c")
    base = wid * _NSLAB // _NW
    end = (wid + 1) * _NSLAB // _NW

    def read(u, slab):
      return pltpu.async_copy(t_h.at[:, pl.ds(u * _SW, _SW)], slab, semr)

    def do_copy(slab, flat, width):
      def cp_body(g, c2):
        for c in range(_DIM):
          flat[pl.ds(c * _SW + g * _LANES, _LANES)] = (
              slab[c, pl.ds(g * _LANES, _LANES)])
        return c2

      lax.fori_loop(0, width // _LANES, cp_body, 0)

    def fire_writes(u, flat, width):
      copies = []
      for c in range(_DIM):
        copies.append(
            pltpu.async_copy(
                flat.at[pl.ds(c * _SW, width)],
                out_h.at[pl.ds(c * _VOCAB + u * _SW, width)], semw))
      return copies

    def pair_body(p, carry):
      u_a = base + 2 * p
      u_b = u_a + 1
      r_a = read(u_a, slab_a)
      r_b = read(u_b, slab_b)
      r_a.wait()
      do_copy(slab_a, flat_a, _SW)
      w_a = fire_writes(u_a, flat_a, _SW)
      r_b.wait()
      do_copy(slab_b, flat_b, _SW)
      w_b = fire_writes(u_b, flat_b, _SW)
      for cp in w_a + w_b:
        cp.wait()
      return carry

    lax.fori_loop(0, 20, pair_body, 0)

    @pl.when(base + 40 < end)
    def _():
      u = base + 40
      read(u, slab_a).wait()
      do_copy(slab_a, flat_a, _SW)
      for cp in fire_writes(u, flat_a, _SW):
        cp.wait()

    @pl.when(wid == 1)
    def _():
      pltpu.sync_copy(tail_h, slab_a.at[:, pl.ds(0, _TAIL)])
      do_copy(slab_a, flat_a, _TAIL)
      copies = []
      for c in range(_DIM):
        copies.append(
            pltpu.async_copy(
                flat_a.at[pl.ds(c * _SW, _TAIL)],
                out_h.at[pl.ds(c * _VOCAB + _TAIL0, _TAIL)], semw))
      for cp in copies:
        cp.wait()

  return k(t_t, t_tail)


def _sc_gather(t_flat, x_t_flat, pos):
  mesh = plsc.VectorSubcoreMesh(core_axis_name="c", subcore_axis_name="s")

  @functools.partial(
      pl.kernel,
      mesh=mesh,
      out_type=jax.ShapeDtypeStruct((_SEQ, _DIM, _BATCH), jnp.float32),
      scratch_types=[
          pltpu.VMEM((_BL,), jnp.int32),
          pltpu.VMEM((_BL,), jnp.int32),
          pltpu.VMEM((_BL,), jnp.float32),
          pltpu.VMEM((_BL,), jnp.float32),
          pltpu.VMEM((_DIM * _BL,), jnp.int32),
          pltpu.VMEM((_DIM * _BL,), jnp.int32),
          pltpu.VMEM((_DIM * _BL,), jnp.float32),
          pltpu.VMEM((_DIM * _BL,), jnp.float32),
          pltpu.VMEM((_DIM,), jnp.float32),
          pltpu.VMEM((_DIM,), jnp.float32),
          pltpu.SemaphoreType.DMA,
          pltpu.SemaphoreType.DMA,
      ],
  )
  def k(t_h, x_h, pos_h, out_h, idx_a, idx_b, keep_a, keep_b, idxall_a,
        idxall_b, cols_a, cols_b, posr_a, posr_b, semg, semw):
    wid = lax.axis_index("s") * 2 + lax.axis_index("c")

    def prep(u, idx_v, keep_v, idxall_v, posr_v):
      s = u // (_BATCH // _BL)
      b0 = (u % (_BATCH // _BL)) * _BL
      pltpu.sync_copy(x_h.at[pl.ds(s * _BATCH + b0, _BL)], idx_v)
      pltpu.sync_copy(pos_h.at[s], posr_v)

      def grp_body(g, c2):
        kv = idx_v[pl.ds(g * _LANES, _LANES)]
        keep_v[pl.ds(g * _LANES, _LANES)] = jnp.where(
            kv != _PAD_IDX, 1.0, 0.0)
        for c in range(_DIM):
          idxall_v[pl.ds(c * _BL + g * _LANES, _LANES)] = kv + c * _VOCAB
        return c2

      lax.fori_loop(0, _NG, grp_body, 0)

    def finish(u, keep_v, cols_v, posr_v):
      s = u // (_BATCH // _BL)
      b0 = (u % (_BATCH // _BL)) * _BL
      pos0 = posr_v[pl.ds(0, _LANES)]
      pos1 = posr_v[pl.ds(_LANES, _LANES)]

      def fix_body(g, c2):
        kf = keep_v[pl.ds(g * _LANES, _LANES)]
        for c in range(_DIM):
          p = pos0[c] if c < _LANES else pos1[c - _LANES]
          cols_v[pl.ds(c * _BL + g * _LANES, _LANES)] = (
              cols_v[pl.ds(c * _BL + g * _LANES, _LANES)] * kf + p)
        return c2

      lax.fori_loop(0, _NG, fix_body, 0)

      out_copies = []
      for c in range(_DIM):
        out_copies.append(
            pltpu.async_copy(
                cols_v.at[pl.ds(c * _BL, _BL)],
                out_h.at[s, c, pl.ds(b0, _BL)], semw))
      for cp in out_copies:
        cp.wait()

    sets = [
        (idx_a, keep_a, idxall_a, cols_a, posr_a),
        (idx_b, keep_b, idxall_b, cols_b, posr_b),
    ]

    def fire(st):
      return pltpu.async_copy(t_h.at[st[2]], st[3], semg)

    handles = []
    for i in range(2):
      st = sets[i]
      prep(i * _NW + wid, st[0], st[1], st[2], st[4])
      handles.append(fire(st))

    for i in range(12):
      st = sets[i % 2]
      handles[i].wait()
      finish(i * _NW + wid, st[1], st[3], st[4])
      if i + 2 < 12:
        prep((i + 2) * _NW + wid, st[0], st[1], st[2], st[4])
        handles.append(fire(st))

    @pl.when(12 * _NW + wid < _NU)
    def _():
      u = 12 * _NW + wid
      prep(u, idx_a, keep_a, idxall_a, posr_a)
      pltpu.async_copy(t_h.at[idxall_a], cols_a, semg).wait()
      finish(u, keep_a, cols_a, posr_a)

  return k(t_flat, x_t_flat, pos)


def kernel(x, embedding_table, pos_embeddings):
  t_t = embedding_table.T
  t_flat = _sc_linearize(t_t, t_t[:, _TAIL0:])
  x_t_flat = x.T.astype(jnp.int32).reshape(-1)
  out = _sc_gather(t_flat, x_t_flat, pos_embeddings)
  return out.transpose(2, 0, 1)


# speedup vs baseline: 1.0052x; 1.0052x over previous
import functools

import jax
import jax.numpy as jnp
from jax import lax
from jax.experimental import pallas as pl
from jax.experimental.pallas import tpu as pltpu
from jax.experimental.pallas import tpu_sc as plsc

_VOCAB = 1000000
_DIM = 32
_BATCH = 4096
_SEQ = 50
_PAD_IDX = 0

_NW = 32
_LANES = 16

_SW = 768
_NSLAB = _VOCAB // _SW
_TAIL = 128
_TAIL0 = _VOCAB - _TAIL

_BL = 512
_NU = _SEQ * (_BATCH // _BL)
_NG = _BL // _LANES


def _sc_linearize(t_t, t_tail):
  mesh = plsc.VectorSubcoreMesh(core_axis_name="c", subcore_axis_name="s")

  @functools.partial(
      pl.kernel,
      mesh=mesh,
      out_type=jax.ShapeDtypeStruct((_DIM * _VOCAB,), jnp.float32),
      scratch_types=[
          pltpu.VMEM((_DIM, _SW), jnp.float32),
          pltpu.VMEM((_DIM, _SW), jnp.float32),
          pltpu.VMEM((_DIM * _SW,), jnp.float32),
          pltpu.VMEM((_DIM * _SW,), jnp.float32),
          pltpu.SemaphoreType.DMA,
          pltpu.SemaphoreType.DMA,
      ],
  )
  def k(t_h, tail_h, out_h, slab_a, slab_b, flat_a, flat_b, semr, semw):
    wid = lax.axis_index("s") * 2 + lax.axis_index("c")
    base = wid * _NSLAB // _NW
    end = (wid + 1) * _NSLAB // _NW

    def read(u, slab):
      return pltpu.async_copy(t_h.at[:, pl.ds(u * _SW, _SW)], slab, semr)

    def do_copy(slab, flat, width):
      def cp_body(g, c2):
        for c in range(_DIM):
          flat[pl.ds(c * _SW + g * _LANES, _LANES)] = (
              slab[c, pl.ds(g * _LANES, _LANES)])
        return c2

      lax.fori_loop(0, width // _LANES, cp_body, 0)

    def fire_writes(u, flat, width):
      copies = []
      for c in range(_DIM):
        copies.append(
            pltpu.async_copy(
                flat.at[pl.ds(c * _SW, width)],
                out_h.at[pl.ds(c * _VOCAB + u * _SW, width)], semw))
      return copies

    def pair_body(p, carry):
      u_a = base + 2 * p
      u_b = u_a + 1
      r_a = read(u_a, slab_a)
      r_b = read(u_b, slab_b)
      r_a.wait()
      do_copy(slab_a, flat_a, _SW)
      w_a = fire_writes(u_a, flat_a, _SW)
      r_b.wait()
      do_copy(slab_b, flat_b, _SW)
      w_b = fire_writes(u_b, flat_b, _SW)
      for cp in w_a + w_b:
        cp.wait()
      return carry

    lax.fori_loop(0, 20, pair_body, 0)

    @pl.when(base + 40 < end)
    def _():
      u = base + 40
      read(u, slab_a).wait()
      do_copy(slab_a, flat_a, _SW)
      for cp in fire_writes(u, flat_a, _SW):
        cp.wait()

    @pl.when(wid == 1)
    def _():
      pltpu.sync_copy(tail_h, slab_a.at[:, pl.ds(0, _TAIL)])
      do_copy(slab_a, flat_a, _TAIL)
      copies = []
      for c in range(_DIM):
        copies.append(
            pltpu.async_copy(
                flat_a.at[pl.ds(c * _SW, _TAIL)],
                out_h.at[pl.ds(c * _VOCAB + _TAIL0, _TAIL)], semw))
      for cp in copies:
        cp.wait()

  return k(t_t, t_tail)


def _sc_gather(t_flat, x_t_flat, pos):
  mesh = plsc.VectorSubcoreMesh(core_axis_name="c", subcore_axis_name="s")

  @functools.partial(
      pl.kernel,
      mesh=mesh,
      out_type=jax.ShapeDtypeStruct((_SEQ, _DIM, _BATCH), jnp.float32),
      scratch_types=[
          pltpu.VMEM((_BL,), jnp.int32),
          pltpu.VMEM((_BL,), jnp.int32),
          pltpu.VMEM((_BL,), jnp.float32),
          pltpu.VMEM((_BL,), jnp.float32),
          pltpu.VMEM((_DIM * _BL,), jnp.int32),
          pltpu.VMEM((_DIM * _BL,), jnp.int32),
          pltpu.VMEM((_DIM * _BL,), jnp.float32),
          pltpu.VMEM((_DIM * _BL,), jnp.float32),
          pltpu.VMEM((_DIM,), jnp.float32),
          pltpu.VMEM((_DIM,), jnp.float32),
          pltpu.SemaphoreType.DMA,
          pltpu.SemaphoreType.DMA,
      ],
  )
  def k(t_h, x_h, pos_h, out_h, idx_a, idx_b, keep_a, keep_b, idxall_a,
        idxall_b, cols_a, cols_b, posr_a, posr_b, semg, semw):
    wid = lax.axis_index("s") * 2 + lax.axis_index("c")

    def prep(u, idx_v, keep_v, idxall_v, posr_v):
      s = u // (_BATCH // _BL)
      b0 = (u % (_BATCH // _BL)) * _BL
      pltpu.sync_copy(x_h.at[pl.ds(s * _BATCH + b0, _BL)], idx_v)
      pltpu.sync_copy(pos_h.at[s], posr_v)

      def grp_body(g, c2):
        kv = idx_v[pl.ds(g * _LANES, _LANES)]
        keep_v[pl.ds(g * _LANES, _LANES)] = jnp.where(
            kv != _PAD_IDX, 1.0, 0.0)
        for c in range(_DIM):
          idxall_v[pl.ds(c * _BL + g * _LANES, _LANES)] = kv + c * _VOCAB
        return c2

      lax.fori_loop(0, _NG, grp_body, 0)

    def finish(u, keep_v, cols_v, posr_v):
      s = u // (_BATCH // _BL)
      b0 = (u % (_BATCH // _BL)) * _BL
      pos0 = posr_v[pl.ds(0, _LANES)]
      pos1 = posr_v[pl.ds(_LANES, _LANES)]

      def fix_body(g, c2):
        kf = keep_v[pl.ds(g * _LANES, _LANES)]
        for c in range(_DIM):
          p = pos0[c] if c < _LANES else pos1[c - _LANES]
          cols_v[pl.ds(c * _BL + g * _LANES, _LANES)] = (
              cols_v[pl.ds(c * _BL + g * _LANES, _LANES)] * kf + p)
        return c2

      lax.fori_loop(0, _NG, fix_body, 0)

      out_copies = []
      for c in range(_DIM):
        out_copies.append(
            pltpu.async_copy(
                cols_v.at[pl.ds(c * _BL, _BL)],
                out_h.at[s, c, pl.ds(b0, _BL)], semw))
      for cp in out_copies:
        cp.wait()

    def pair_body(p, carry):
      u_a = (2 * p) * _NW + wid
      u_b = u_a + _NW
      prep(u_a, idx_a, keep_a, idxall_a, posr_a)
      g_a = pltpu.async_copy(t_h.at[idxall_a], cols_a, semg)
      prep(u_b, idx_b, keep_b, idxall_b, posr_b)
      g_b = pltpu.async_copy(t_h.at[idxall_b], cols_b, semg)
      g_a.wait()
      finish(u_a, keep_a, cols_a, posr_a)
      g_b.wait()
      finish(u_b, keep_b, cols_b, posr_b)
      return carry

    lax.fori_loop(0, 6, pair_body, 0)

    @pl.when(12 * _NW + wid < _NU)
    def _():
      u = 12 * _NW + wid
      prep(u, idx_a, keep_a, idxall_a, posr_a)
      pltpu.async_copy(t_h.at[idxall_a], cols_a, semg).wait()
      finish(u, keep_a, cols_a, posr_a)

  return k(t_flat, x_t_flat, pos)


def kernel(x, embedding_table, pos_embeddings):
  t_t = embedding_table.T
  t_flat = _sc_linearize(t_t, t_t[:, _TAIL0:])
  x_t_flat = x.T.astype(jnp.int32).reshape(-1)
  out = _sc_gather(t_flat, x_t_flat, pos_embeddings)
  return out.transpose(2, 0, 1)
